# Initial kernel scaffold; baseline (speedup 1.0000x reference)
import jax
import jax.numpy as jnp
from jax.experimental import pallas as pl


def kernel(sim_mm_f, sim_mm_s, sim_mm_g, sim_dd_t, sim_dd_s, sim_dd_g, all_node_feat, m_node, d_node, mi_emb, dj_emb, pair_confidence, W_mm_f, b_mm_f, W_mm_s, b_mm_s, W_mm_g, b_mm_g, W_dd_t, b_dd_t, W_dd_s, b_dd_s, W_dd_g, b_dd_g, Wq_m, bq_m, Wk_m, bk_m, Wv_m, bv_m, Wo_m, bo_m, Wq_d, bq_d, Wk_d, bk_d, Wv_d, bv_d, Wo_d, bo_d, W1, b1, W2, b2):
    raise NotImplementedError("write your pallas kernel here")


# speedup vs baseline: 16.4938x; 16.4938x over previous
import functools

import jax
import jax.numpy as jnp
from jax.experimental import pallas as pl

_M = 4096
_FEAT = 128
_TOPK = 32
_ROW_BLK = 256
_B_BLK = 256


def _agg_kernel(sim_ref, feats_ref, wt_ref, b_ref, out_ref, *, topk, row_blk):
    pid = pl.program_id(0)
    v = sim_ref[...]
    rows = jax.lax.broadcasted_iota(jnp.int32, v.shape, 0) + pid * row_blk
    cols = jax.lax.broadcasted_iota(jnp.int32, v.shape, 1)
    v = jnp.where(rows == cols, 0.0, v)
    keys = jax.lax.bitcast_convert_type(v, jnp.int32)
    t = jnp.zeros((v.shape[0], 1), jnp.int32)
    for bit in range(30, -1, -1):
        cand = t | (1 << bit)
        cnt = jnp.sum((keys >= cand).astype(jnp.int32), axis=1, keepdims=True)
        t = jnp.where(cnt >= topk, cand, t)
    gt = keys > t
    eq = keys == t
    need = topk - jnp.sum(gt.astype(jnp.int32), axis=1, keepdims=True)
    c = jnp.zeros((v.shape[0], 1), jnp.int32)
    for bit in range(12, -1, -1):
        cand = c | (1 << bit)
        cntc = jnp.sum((eq & (cols < cand)).astype(jnp.int32), axis=1,
                       keepdims=True)
        c = jnp.where(cntc <= need, cand, c)
    mask = gt | (eq & (cols < c))
    wv = jnp.where(mask, v, 0.0)
    s = jnp.sum(wv, axis=1, keepdims=True)
    wn = wv / jnp.maximum(s, 1e-8)
    token = jnp.dot(wn, feats_ref[...], preferred_element_type=jnp.float32)
    lin = jnp.dot(token, wt_ref[...], preferred_element_type=jnp.float32)
    out_ref[...] = jnp.maximum(lin + b_ref[...], 0.0)


def _agg(sim, feats, w, b, *, topk=_TOPK, row_blk=_ROW_BLK):
    n = sim.shape[0]
    feat = feats.shape[1]
    body = functools.partial(_agg_kernel, topk=topk, row_blk=row_blk)
    return pl.pallas_call(
        body,
        grid=(n // row_blk,),
        in_specs=[
            pl.BlockSpec((row_blk, n), lambda i: (i, 0)),
            pl.BlockSpec((n, feat), lambda i: (0, 0)),
            pl.BlockSpec((feat, feat), lambda i: (0, 0)),
            pl.BlockSpec((1, feat), lambda i: (0, 0)),
        ],
        out_specs=pl.BlockSpec((row_blk, feat), lambda i: (i, 0)),
        out_shape=jax.ShapeDtypeStruct((n, feat), jnp.float32),
    )(sim, feats, w.T, b.reshape(1, feat))


def _softmax3(l1, l2, l3):
    mx = jnp.maximum(jnp.maximum(l1, l2), l3)
    e1 = jnp.exp(l1 - mx)
    e2 = jnp.exp(l2 - mx)
    e3 = jnp.exp(l3 - mx)
    den = e1 + e2 + e3
    return e1 / den, e2 / den, e3 / den


def _fusion(toks, emb, wqt, bq, wkt, bk, wvt, bv, wot, bo, feat):
    q = jnp.dot(emb, wqt, preferred_element_type=jnp.float32) + bq
    logits = []
    vals = []
    for tk in toks:
        kk = jnp.dot(tk, wkt, preferred_element_type=jnp.float32) + bk
        logits.append(jnp.sum(q * kk, axis=1, keepdims=True) / (feat ** 0.5))
        vals.append(jnp.dot(tk, wvt, preferred_element_type=jnp.float32) + bv)
    a1, a2, a3 = _softmax3(*logits)
    fused = a1 * vals[0] + a2 * vals[1] + a3 * vals[2]
    mean = (toks[0] + toks[1] + toks[2]) / 3.0
    var = ((toks[0] - mean) ** 2 + (toks[1] - mean) ** 2
           + (toks[2] - mean) ** 2) / 2.0
    dis = jnp.sqrt(var)
    out = (jnp.dot(fused, wot[:feat], preferred_element_type=jnp.float32)
           + jnp.dot(dis, wot[feat:], preferred_element_type=jnp.float32) + bo)
    return jnp.maximum(out, 0.0)


def _head_kernel(tmf_ref, tms_ref, tmg_ref, tdt_ref, tds_ref, tdg_ref,
                 mn_ref, dn_ref, mi_ref, dj_ref, pc_ref,
                 wqmt_ref, bqm_ref, wkmt_ref, bkm_ref, wvmt_ref, bvm_ref,
                 womt_ref, bom_ref,
                 wqdt_ref, bqd_ref, wkdt_ref, bkd_ref, wvdt_ref, bvd_ref,
                 wodt_ref, bod_ref,
                 w1t_ref, b1_ref, w2t_ref, b2_ref, out_ref,
                 *, b_blk, feat):
    pid = pl.program_id(0)
    n = tmf_ref.shape[0]
    mn = mn_ref[pl.ds(pid * b_blk, b_blk)].reshape(b_blk, 1)
    dn = dn_ref[pl.ds(pid * b_blk, b_blk)].reshape(b_blk, 1)
    pc = pc_ref[pl.ds(pid * b_blk, b_blk)].reshape(b_blk, 1)
    cols = jax.lax.broadcasted_iota(jnp.int32, (b_blk, n), 1)
    oh_m = (mn == cols).astype(jnp.float32)
    oh_d = (dn == cols).astype(jnp.float32)
    m_toks = [jnp.dot(oh_m, r[...], preferred_element_type=jnp.float32)
              for r in (tmf_ref, tms_ref, tmg_ref)]
    d_toks = [jnp.dot(oh_d, r[...], preferred_element_type=jnp.float32)
              for r in (tdt_ref, tds_ref, tdg_ref)]
    mi = mi_ref[...]
    dj = dj_ref[...]
    mi_sem = _fusion(m_toks, mi, wqmt_ref[...], bqm_ref[...], wkmt_ref[...],
                     bkm_ref[...], wvmt_ref[...], bvm_ref[...], womt_ref[...],
                     bom_ref[...], feat)
    dj_sem = _fusion(d_toks, dj, wqdt_ref[...], bqd_ref[...], wkdt_ref[...],
                     bkd_ref[...], wvdt_ref[...], bvd_ref[...], wodt_ref[...],
                     bod_ref[...], feat)
    w1t = w1t_ref[...]
    parts = [mi, dj, mi_sem, dj_sem, jnp.abs(mi_sem - dj_sem),
             mi_sem * dj_sem]
    acc = b1_ref[...] + pc * w1t[6 * feat:]
    for i, p in enumerate(parts):
        acc = acc + jnp.dot(p, w1t[i * feat:(i + 1) * feat],
                            preferred_element_type=jnp.float32)
    h = jnp.maximum(acc, 0.0)
    out = jnp.dot(h, w2t_ref[...], preferred_element_type=jnp.float32)
    out_ref[...] = jnp.maximum(out + b2_ref[...], 0.0)


def _head(toks_m, toks_d, m_node, d_node, mi_emb, dj_emb, pc,
          wq_m, bq_m, wk_m, bk_m, wv_m, bv_m, wo_m, bo_m,
          wq_d, bq_d, wk_d, bk_d, wv_d, bv_d, wo_d, bo_d,
          w1, b1, w2, b2, *, b_blk=_B_BLK):
    bsz = m_node.shape[0]
    n = toks_m[0].shape[0]
    feat = toks_m[0].shape[1]
    hid = w1.shape[0]
    body = functools.partial(_head_kernel, b_blk=b_blk, feat=feat)
    full = lambda shape: pl.BlockSpec(shape, lambda i: tuple(0 for _ in shape))
    in_specs = (
        [full((n, feat))] * 6
        + [full((bsz,)), full((bsz,))]
        + [pl.BlockSpec((b_blk, feat), lambda i: (i, 0))] * 2
        + [full((bsz,))]
        + [full((feat, feat)), full((1, feat))] * 3
        + [full((2 * feat, feat)), full((1, feat))]
        + [full((feat, feat)), full((1, feat))] * 3
        + [full((2 * feat, feat)), full((1, feat))]
        + [full((6 * feat + 1, hid)), full((1, hid))]
        + [full((hid, hid)), full((1, hid))]
    )
    args = (
        list(toks_m) + list(toks_d)
        + [m_node.astype(jnp.int32), d_node.astype(jnp.int32),
           mi_emb, dj_emb, pc]
        + [wq_m.T, bq_m.reshape(1, feat), wk_m.T, bk_m.reshape(1, feat),
           wv_m.T, bv_m.reshape(1, feat), wo_m.T, bo_m.reshape(1, feat)]
        + [wq_d.T, bq_d.reshape(1, feat), wk_d.T, bk_d.reshape(1, feat),
           wv_d.T, bv_d.reshape(1, feat), wo_d.T, bo_d.reshape(1, feat)]
        + [w1.T, b1.reshape(1, hid), w2.T, b2.reshape(1, hid)]
    )
    return pl.pallas_call(
        body,
        grid=(bsz // b_blk,),
        in_specs=in_specs,
        out_specs=pl.BlockSpec((b_blk, hid), lambda i: (i, 0)),
        out_shape=jax.ShapeDtypeStruct((bsz, hid), jnp.float32),
    )(*args)


def kernel(sim_mm_f, sim_mm_s, sim_mm_g, sim_dd_t, sim_dd_s, sim_dd_g,
           all_node_feat, m_node, d_node, mi_emb, dj_emb, pair_confidence,
           W_mm_f, b_mm_f, W_mm_s, b_mm_s, W_mm_g, b_mm_g,
           W_dd_t, b_dd_t, W_dd_s, b_dd_s, W_dd_g, b_dd_g,
           Wq_m, bq_m, Wk_m, bk_m, Wv_m, bv_m, Wo_m, bo_m,
           Wq_d, bq_d, Wk_d, bk_d, Wv_d, bv_d, Wo_d, bo_d,
           W1, b1, W2, b2):
    m = sim_mm_f.shape[0]
    mi_feat = all_node_feat[:m]
    di_feat = all_node_feat[m:]
    toks_m = [
        _agg(sim_mm_f, mi_feat, W_mm_f, b_mm_f),
        _agg(sim_mm_s, mi_feat, W_mm_s, b_mm_s),
        _agg(sim_mm_g, mi_feat, W_mm_g, b_mm_g),
    ]
    toks_d = [
        _agg(sim_dd_t, di_feat, W_dd_t, b_dd_t),
        _agg(sim_dd_s, di_feat, W_dd_s, b_dd_s),
        _agg(sim_dd_g, di_feat, W_dd_g, b_dd_g),
    ]
    return _head(toks_m, toks_d, m_node, d_node, mi_emb, dj_emb,
                 pair_confidence,
                 Wq_m, bq_m, Wk_m, bk_m, Wv_m, bv_m, Wo_m, bo_m,
                 Wq_d, bq_d, Wk_d, bk_d, Wv_d, bv_d, Wo_d, bo_d,
                 W1, b1, W2, b2)


# speedup vs baseline: 20.8823x; 1.2661x over previous
import functools

import jax
import jax.numpy as jnp
from jax.experimental import pallas as pl

_M = 4096
_FEAT = 128
_TOPK = 32
_ROW_BLK = 256
_B_BLK = 256


def _agg_kernel(sim_ref, feats_ref, wt_ref, b_ref, out_ref, *, topk, row_blk):
    pid = pl.program_id(0)
    v = sim_ref[...]
    rows = jax.lax.broadcasted_iota(jnp.int32, v.shape, 0) + pid * row_blk
    cols = jax.lax.broadcasted_iota(jnp.int32, v.shape, 1)
    v = jnp.where(rows == cols, 0.0, v)
    keys = jax.lax.bitcast_convert_type(v, jnp.int32)
    n = v.shape[1]
    h = n // 2
    kl = keys[:, :h]
    kr = keys[:, h:]
    guard = jnp.int32(-2147450880)

    def swar_ge_count(x, cand):
        y = cand * 0x00010001
        g = ((x - y) >> 15) & 0x00010001
        s = jnp.sum(g, axis=1, keepdims=True)
        return (s & 0xFFFF) + (s >> 16)

    xh = (((kl << 1) & jnp.int32(-65536)) | (kr >> 15)) | guard
    t = jnp.zeros((v.shape[0], 1), jnp.int32)
    for bit in range(14, -1, -1):
        cand = t | (1 << bit)
        cnt = swar_ge_count(xh, cand)
        t = jnp.where(cnt >= topk, cand, t)
    gt_hi = swar_ge_count(xh, t + 1)
    need1 = topk - gt_hi

    eq_l = (kl >> 15) == t
    eq_r = (kr >> 15) == t
    xl = ((jnp.where(eq_l, kl & 0x7FFF, 0) << 16)
          | jnp.where(eq_r, kr & 0x7FFF, 0)) | guard
    tl = jnp.zeros((v.shape[0], 1), jnp.int32)
    for bit in range(14, -1, -1):
        cand = tl | (1 << bit)
        cnt = swar_ge_count(xl, cand)
        tl = jnp.where(cnt >= need1, cand, tl)
    need = topk - gt_hi - swar_ge_count(xl, tl + 1)

    tfull = (t << 15) | tl
    gt = keys > tfull
    eq = keys == tfull
    cidx = jax.lax.broadcasted_iota(jnp.int32, (v.shape[0], h), 1)
    xc = ((jnp.where(eq[:, :h], cidx, 0x7FFF) << 16)
          | jnp.where(eq[:, h:], cidx + h, 0x7FFF)) | guard
    c = jnp.zeros((v.shape[0], 1), jnp.int32)
    for bit in range(12, -1, -1):
        cand = c | (1 << bit)
        cnt_lt = n - swar_ge_count(xc, cand)
        c = jnp.where(cnt_lt <= need, cand, c)
    mask = gt | (eq & (cols < c))
    wv = jnp.where(mask, v, 0.0)
    s = jnp.sum(wv, axis=1, keepdims=True)
    wn = wv / jnp.maximum(s, 1e-8)
    token = jnp.dot(wn, feats_ref[...], preferred_element_type=jnp.float32)
    lin = jnp.dot(token, wt_ref[...], preferred_element_type=jnp.float32)
    out_ref[...] = jnp.maximum(lin + b_ref[...], 0.0)


def _agg(sim, feats, w, b, *, topk=_TOPK, row_blk=_ROW_BLK):
    n = sim.shape[0]
    feat = feats.shape[1]
    body = functools.partial(_agg_kernel, topk=topk, row_blk=row_blk)
    return pl.pallas_call(
        body,
        grid=(n // row_blk,),
        in_specs=[
            pl.BlockSpec((row_blk, n), lambda i: (i, 0)),
            pl.BlockSpec((n, feat), lambda i: (0, 0)),
            pl.BlockSpec((feat, feat), lambda i: (0, 0)),
            pl.BlockSpec((1, feat), lambda i: (0, 0)),
        ],
        out_specs=pl.BlockSpec((row_blk, feat), lambda i: (i, 0)),
        out_shape=jax.ShapeDtypeStruct((n, feat), jnp.float32),
    )(sim, feats, w.T, b.reshape(1, feat))


def _softmax3(l1, l2, l3):
    mx = jnp.maximum(jnp.maximum(l1, l2), l3)
    e1 = jnp.exp(l1 - mx)
    e2 = jnp.exp(l2 - mx)
    e3 = jnp.exp(l3 - mx)
    den = e1 + e2 + e3
    return e1 / den, e2 / den, e3 / den


def _fusion(toks, emb, wqt, bq, wkt, bk, wvt, bv, wot, bo, feat):
    q = jnp.dot(emb, wqt, preferred_element_type=jnp.float32) + bq
    logits = []
    vals = []
    for tk in toks:
        kk = jnp.dot(tk, wkt, preferred_element_type=jnp.float32) + bk
        logits.append(jnp.sum(q * kk, axis=1, keepdims=True) / (feat ** 0.5))
        vals.append(jnp.dot(tk, wvt, preferred_element_type=jnp.float32) + bv)
    a1, a2, a3 = _softmax3(*logits)
    fused = a1 * vals[0] + a2 * vals[1] + a3 * vals[2]
    mean = (toks[0] + toks[1] + toks[2]) / 3.0
    var = ((toks[0] - mean) ** 2 + (toks[1] - mean) ** 2
           + (toks[2] - mean) ** 2) / 2.0
    dis = jnp.sqrt(var)
    out = (jnp.dot(fused, wot[:feat], preferred_element_type=jnp.float32)
           + jnp.dot(dis, wot[feat:], preferred_element_type=jnp.float32) + bo)
    return jnp.maximum(out, 0.0)


def _head_kernel(tmf_ref, tms_ref, tmg_ref, tdt_ref, tds_ref, tdg_ref,
                 mn_ref, dn_ref, mi_ref, dj_ref, pc_ref,
                 wqmt_ref, bqm_ref, wkmt_ref, bkm_ref, wvmt_ref, bvm_ref,
                 womt_ref, bom_ref,
                 wqdt_ref, bqd_ref, wkdt_ref, bkd_ref, wvdt_ref, bvd_ref,
                 wodt_ref, bod_ref,
                 w1t_ref, b1_ref, w2t_ref, b2_ref, out_ref,
                 *, b_blk, feat):
    pid = pl.program_id(0)
    n = tmf_ref.shape[0]
    mn = mn_ref[pl.ds(pid * b_blk, b_blk)].reshape(b_blk, 1)
    dn = dn_ref[pl.ds(pid * b_blk, b_blk)].reshape(b_blk, 1)
    pc = pc_ref[pl.ds(pid * b_blk, b_blk)].reshape(b_blk, 1)
    cols = jax.lax.broadcasted_iota(jnp.int32, (b_blk, n), 1)
    oh_m = (mn == cols).astype(jnp.float32)
    oh_d = (dn == cols).astype(jnp.float32)
    m_toks = [jnp.dot(oh_m, r[...], preferred_element_type=jnp.float32)
              for r in (tmf_ref, tms_ref, tmg_ref)]
    d_toks = [jnp.dot(oh_d, r[...], preferred_element_type=jnp.float32)
              for r in (tdt_ref, tds_ref, tdg_ref)]
    mi = mi_ref[...]
    dj = dj_ref[...]
    mi_sem = _fusion(m_toks, mi, wqmt_ref[...], bqm_ref[...], wkmt_ref[...],
                     bkm_ref[...], wvmt_ref[...], bvm_ref[...], womt_ref[...],
                     bom_ref[...], feat)
    dj_sem = _fusion(d_toks, dj, wqdt_ref[...], bqd_ref[...], wkdt_ref[...],
                     bkd_ref[...], wvdt_ref[...], bvd_ref[...], wodt_ref[...],
                     bod_ref[...], feat)
    w1t = w1t_ref[...]
    parts = [mi, dj, mi_sem, dj_sem, jnp.abs(mi_sem - dj_sem),
             mi_sem * dj_sem]
    acc = b1_ref[...] + pc * w1t[6 * feat:]
    for i, p in enumerate(parts):
        acc = acc + jnp.dot(p, w1t[i * feat:(i + 1) * feat],
                            preferred_element_type=jnp.float32)
    h = jnp.maximum(acc, 0.0)
    out = jnp.dot(h, w2t_ref[...], preferred_element_type=jnp.float32)
    out_ref[...] = jnp.maximum(out + b2_ref[...], 0.0)


def _head(toks_m, toks_d, m_node, d_node, mi_emb, dj_emb, pc,
          wq_m, bq_m, wk_m, bk_m, wv_m, bv_m, wo_m, bo_m,
          wq_d, bq_d, wk_d, bk_d, wv_d, bv_d, wo_d, bo_d,
          w1, b1, w2, b2, *, b_blk=_B_BLK):
    bsz = m_node.shape[0]
    n = toks_m[0].shape[0]
    feat = toks_m[0].shape[1]
    hid = w1.shape[0]
    body = functools.partial(_head_kernel, b_blk=b_blk, feat=feat)
    full = lambda shape: pl.BlockSpec(shape, lambda i: tuple(0 for _ in shape))
    in_specs = (
        [full((n, feat))] * 6
        + [full((bsz,)), full((bsz,))]
        + [pl.BlockSpec((b_blk, feat), lambda i: (i, 0))] * 2
        + [full((bsz,))]
        + [full((feat, feat)), full((1, feat))] * 3
        + [full((2 * feat, feat)), full((1, feat))]
        + [full((feat, feat)), full((1, feat))] * 3
        + [full((2 * feat, feat)), full((1, feat))]
        + [full((6 * feat + 1, hid)), full((1, hid))]
        + [full((hid, hid)), full((1, hid))]
    )
    args = (
        list(toks_m) + list(toks_d)
        + [m_node.astype(jnp.int32), d_node.astype(jnp.int32),
           mi_emb, dj_emb, pc]
        + [wq_m.T, bq_m.reshape(1, feat), wk_m.T, bk_m.reshape(1, feat),
           wv_m.T, bv_m.reshape(1, feat), wo_m.T, bo_m.reshape(1, feat)]
        + [wq_d.T, bq_d.reshape(1, feat), wk_d.T, bk_d.reshape(1, feat),
           wv_d.T, bv_d.reshape(1, feat), wo_d.T, bo_d.reshape(1, feat)]
        + [w1.T, b1.reshape(1, hid), w2.T, b2.reshape(1, hid)]
    )
    return pl.pallas_call(
        body,
        grid=(bsz // b_blk,),
        in_specs=in_specs,
        out_specs=pl.BlockSpec((b_blk, hid), lambda i: (i, 0)),
        out_shape=jax.ShapeDtypeStruct((bsz, hid), jnp.float32),
    )(*args)


def kernel(sim_mm_f, sim_mm_s, sim_mm_g, sim_dd_t, sim_dd_s, sim_dd_g,
           all_node_feat, m_node, d_node, mi_emb, dj_emb, pair_confidence,
           W_mm_f, b_mm_f, W_mm_s, b_mm_s, W_mm_g, b_mm_g,
           W_dd_t, b_dd_t, W_dd_s, b_dd_s, W_dd_g, b_dd_g,
           Wq_m, bq_m, Wk_m, bk_m, Wv_m, bv_m, Wo_m, bo_m,
           Wq_d, bq_d, Wk_d, bk_d, Wv_d, bv_d, Wo_d, bo_d,
           W1, b1, W2, b2):
    m = sim_mm_f.shape[0]
    mi_feat = all_node_feat[:m]
    di_feat = all_node_feat[m:]
    toks_m = [
        _agg(sim_mm_f, mi_feat, W_mm_f, b_mm_f),
        _agg(sim_mm_s, mi_feat, W_mm_s, b_mm_s),
        _agg(sim_mm_g, mi_feat, W_mm_g, b_mm_g),
    ]
    toks_d = [
        _agg(sim_dd_t, di_feat, W_dd_t, b_dd_t),
        _agg(sim_dd_s, di_feat, W_dd_s, b_dd_s),
        _agg(sim_dd_g, di_feat, W_dd_g, b_dd_g),
    ]
    return _head(toks_m, toks_d, m_node, d_node, mi_emb, dj_emb,
                 pair_confidence,
                 Wq_m, bq_m, Wk_m, bk_m, Wv_m, bv_m, Wo_m, bo_m,
                 Wq_d, bq_d, Wk_d, bk_d, Wv_d, bv_d, Wo_d, bo_d,
                 W1, b1, W2, b2)
